# Initial kernel scaffold; baseline (speedup 1.0000x reference)
import jax
import jax.numpy as jnp
from jax.experimental import pallas as pl


def kernel(x, index):
    raise NotImplementedError("write your pallas kernel here")


# speedup vs baseline: 25.6424x; 25.6424x over previous
import functools

import jax
import jax.numpy as jnp
from jax import lax
from jax.experimental import pallas as pl
from jax.experimental.pallas import tpu as pltpu
from jax.experimental.pallas import tpu_sc as plsc

E = 6400000
N = 100000
NPAD = 100352
NW = 32
E_PER_W = E // NW
BLK = 4000
NBLK = E_PER_W // BLK
VEC = 16
UNROLL = 5


def _sc_partial_sums(x, index):
  mesh = plsc.VectorSubcoreMesh(core_axis_name="c", subcore_axis_name="s")

  @functools.partial(
      pl.kernel,
      mesh=mesh,
      out_type=jax.ShapeDtypeStruct((NW, NPAD), jnp.float32),
      compiler_params=pltpu.CompilerParams(needs_layout_passes=False),
      scratch_types=[
          pltpu.VMEM((NPAD,), jnp.float32),
          pltpu.VMEM((BLK,), jnp.float32),
          pltpu.VMEM((BLK,), jnp.float32),
          pltpu.VMEM((BLK,), jnp.int32),
          pltpu.VMEM((BLK,), jnp.int32),
          pltpu.SemaphoreType.DMA,
          pltpu.SemaphoreType.DMA,
          pltpu.SemaphoreType.DMA,
          pltpu.SemaphoreType.DMA,
      ],
  )
  def k(x_hbm, idx_hbm, out_hbm, acc, xb0, xb1, ib0, ib1, sx0, sx1, si0, si1):
    cid = lax.axis_index("c")
    sid = lax.axis_index("s")
    wid = sid * 2 + cid
    base = wid * E_PER_W

    xb = (xb0, xb1)
    ib = (ib0, ib1)
    sx = (sx0, sx1)
    si = (si0, si1)

    zeros = jnp.zeros((VEC,), jnp.float32)

    def zbody(i, c):
      for u in range(4):
        acc[pl.ds((i * 4 + u) * VEC, VEC)] = zeros
      return c

    lax.fori_loop(0, NPAD // (VEC * 4), zbody, 0)

    def make_copies(blk, slot):
      off = base + blk * BLK
      cpx = pltpu.make_async_copy(
          x_hbm.at[pl.ds(off, BLK)], xb[slot], sx[slot])
      cpi = pltpu.make_async_copy(
          idx_hbm.at[pl.ds(off, BLK)], ib[slot], si[slot])
      return cpx, cpi

    def start(blk, slot):
      cpx, cpi = make_copies(blk, slot)
      cpx.start()
      cpi.start()

    def wait(blk, slot):
      cpx, cpi = make_copies(blk, slot)
      cpx.wait()
      cpi.wait()

    def process(slot):
      xbuf = xb[slot]
      ibuf = ib[slot]

      def ibody(i, c):
        for u in range(UNROLL):
          o = (i * UNROLL + u) * VEC
          idxv = ibuf[pl.ds(o, VEC)]
          xv = xbuf[pl.ds(o, VEC)]
          plsc.addupdate_scatter(acc, [idxv], xv)
        return c

      lax.fori_loop(0, BLK // (VEC * UNROLL), ibody, 0)

    start(0, 0)
    start(1, 1)
    for blk in range(NBLK):
      slot = blk % 2
      wait(blk, slot)
      process(slot)
      if blk + 2 < NBLK:
        start(blk + 2, slot)

    pltpu.sync_copy(acc, out_hbm.at[wid])

  return k(x, index)


def _tc_reduce(partials):
  def body(p_ref, o_ref):
    o_ref[...] = jnp.sum(p_ref[...], axis=0)

  return pl.pallas_call(
      body,
      out_shape=jax.ShapeDtypeStruct((NPAD,), jnp.float32),
  )(partials)


@jax.jit
def kernel(x, index):
  partials = _sc_partial_sums(x, index)
  out = _tc_reduce(partials)
  return out[:N]


# speedup vs baseline: 30.2576x; 1.1800x over previous
import functools

import jax
import jax.numpy as jnp
from jax import lax
from jax.experimental import pallas as pl
from jax.experimental.pallas import tpu as pltpu
from jax.experimental.pallas import tpu_sc as plsc

E = 6400000
N = 100000
NPAD = 100352
NW = 32
E_PER_W = E // NW
BLK = 4000
NBLK = E_PER_W // BLK
VEC = 16
UNROLL = 5


def _sc_partial_sums(x, index):
  mesh = plsc.VectorSubcoreMesh(core_axis_name="c", subcore_axis_name="s")

  @functools.partial(
      pl.kernel,
      mesh=mesh,
      out_type=jax.ShapeDtypeStruct((NW, NPAD), jnp.float32),
      compiler_params=pltpu.CompilerParams(needs_layout_passes=False),
      scratch_types=[
          pltpu.VMEM((NPAD,), jnp.float32),
          pltpu.VMEM((BLK,), jnp.float32),
          pltpu.VMEM((BLK,), jnp.float32),
          pltpu.VMEM((BLK,), jnp.int32),
          pltpu.VMEM((BLK,), jnp.int32),
          pltpu.SemaphoreType.DMA,
          pltpu.SemaphoreType.DMA,
          pltpu.SemaphoreType.DMA,
          pltpu.SemaphoreType.DMA,
      ],
  )
  def k(x_hbm, idx_hbm, out_hbm, acc, xb0, xb1, ib0, ib1, sx0, sx1, si0, si1):
    cid = lax.axis_index("c")
    sid = lax.axis_index("s")
    wid = sid * 2 + cid
    base = wid * E_PER_W

    xb = (xb0, xb1)
    ib = (ib0, ib1)
    sx = (sx0, sx1)
    si = (si0, si1)

    zeros = jnp.zeros((VEC,), jnp.float32)

    @plsc.parallel_loop(0, NPAD // VEC, unroll=8)
    def _zero(i):
      acc[pl.ds(i * VEC, VEC)] = zeros

    def make_copies(blk, slot):
      off = base + blk * BLK
      cpx = pltpu.make_async_copy(
          x_hbm.at[pl.ds(off, BLK)], xb[slot], sx[slot])
      cpi = pltpu.make_async_copy(
          idx_hbm.at[pl.ds(off, BLK)], ib[slot], si[slot])
      return cpx, cpi

    def start(blk, slot):
      cpx, cpi = make_copies(blk, slot)
      cpx.start()
      cpi.start()

    def wait(blk, slot):
      cpx, cpi = make_copies(blk, slot)
      cpx.wait()
      cpi.wait()

    def process(slot):
      xbuf = xb[slot]
      ibuf = ib[slot]

      @plsc.parallel_loop(0, BLK // VEC, unroll=UNROLL)
      def _scatter(i):
        o = i * VEC
        idxv = ibuf[pl.ds(o, VEC)]
        xv = xbuf[pl.ds(o, VEC)]
        plsc.addupdate_scatter(acc, [idxv], xv)

    start(0, 0)
    start(1, 1)
    for blk in range(NBLK):
      slot = blk % 2
      wait(blk, slot)
      process(slot)
      if blk + 2 < NBLK:
        start(blk + 2, slot)

    pltpu.sync_copy(acc, out_hbm.at[wid])

  return k(x, index)


def _tc_reduce(partials):
  def body(p_ref, o_ref):
    o_ref[...] = jnp.sum(p_ref[...], axis=0)

  return pl.pallas_call(
      body,
      out_shape=jax.ShapeDtypeStruct((NPAD,), jnp.float32),
  )(partials)


@jax.jit
def kernel(x, index):
  partials = _sc_partial_sums(x, index)
  out = _tc_reduce(partials)
  return out[:N]


# speedup vs baseline: 77.5431x; 2.5628x over previous
import functools

import jax
import jax.numpy as jnp
from jax import lax
from jax.experimental import pallas as pl
from jax.experimental.pallas import tpu as pltpu
from jax.experimental.pallas import tpu_sc as plsc

E = 6400000
N = 100000
NPAD = 100352
NW = 32
E_PER_W = E // NW
BLK = 2000
NBLK = E_PER_W // BLK
VEC = 16
SUB = BLK // VEC
UNROLL = 5


def _sc_partial_sums(x, index):
  mesh = plsc.VectorSubcoreMesh(core_axis_name="c", subcore_axis_name="s")

  @functools.partial(
      pl.kernel,
      mesh=mesh,
      out_type=jax.ShapeDtypeStruct((NW, NPAD), jnp.float32),
      compiler_params=pltpu.CompilerParams(needs_layout_passes=False),
      scratch_types=[
          pltpu.VMEM((NPAD,), jnp.float32),
          pltpu.VMEM((BLK,), jnp.float32),
          pltpu.VMEM((BLK,), jnp.float32),
          pltpu.VMEM((BLK,), jnp.int32),
          pltpu.VMEM((BLK,), jnp.int32),
          pltpu.SemaphoreType.DMA,
          pltpu.SemaphoreType.DMA,
          pltpu.SemaphoreType.DMA,
          pltpu.SemaphoreType.DMA,
      ],
  )
  def k(x_hbm, idx_hbm, out_hbm, acc, xb0, xb1, ib0, ib1, sx0, sx1, si0, si1):
    cid = lax.axis_index("c")
    sid = lax.axis_index("s")
    wid = sid * 2 + cid
    base = wid * E_PER_W

    xb = (xb0, xb1)
    ib = (ib0, ib1)
    sx = (sx0, sx1)
    si = (si0, si1)

    zeros = jnp.zeros((VEC,), jnp.float32)

    @plsc.parallel_loop(0, NPAD // VEC, unroll=8)
    def _zero(i):
      acc[pl.ds(i * VEC, VEC)] = zeros

    def make_copies(blk, slot):
      off = base + blk * BLK
      cpx = pltpu.make_async_copy(
          x_hbm.at[pl.ds(off, BLK)], xb[slot], sx[slot])
      cpi = pltpu.make_async_copy(
          idx_hbm.at[pl.ds(off, BLK)], ib[slot], si[slot])
      return cpx, cpi

    def start(blk, slot):
      cpx, cpi = make_copies(blk, slot)
      cpx.start()
      cpi.start()

    def wait(blk, slot):
      cpx, cpi = make_copies(blk, slot)
      cpx.wait()
      cpi.wait()

    lanebase = jnp.arange(VEC, dtype=jnp.int32) * SUB

    def process(slot):
      xbuf = xb[slot]
      ibuf = ib[slot]

      @plsc.parallel_loop(0, SUB, unroll=UNROLL)
      def _scatter(i):
        pos = lanebase + i
        idxv = plsc.load_gather(ibuf, [pos])
        xv = plsc.load_gather(xbuf, [pos])
        plsc.addupdate_scatter(acc, [idxv], xv)

    start(0, 0)
    start(1, 1)
    for blk in range(NBLK):
      slot = blk % 2
      wait(blk, slot)
      process(slot)
      if blk + 2 < NBLK:
        start(blk + 2, slot)

    pltpu.sync_copy(acc, out_hbm.at[wid])

  return k(x, index)


def _tc_reduce(partials):
  def body(p_ref, o_ref):
    o_ref[...] = jnp.sum(p_ref[...], axis=0)

  return pl.pallas_call(
      body,
      out_shape=jax.ShapeDtypeStruct((NPAD,), jnp.float32),
  )(partials)


@jax.jit
def kernel(x, index):
  partials = _sc_partial_sums(x, index)
  out = _tc_reduce(partials)
  return out[:N]


# speedup vs baseline: 101.7277x; 1.3119x over previous
import functools

import jax
import jax.numpy as jnp
from jax import lax
from jax.experimental import pallas as pl
from jax.experimental.pallas import tpu as pltpu
from jax.experimental.pallas import tpu_sc as plsc

E = 6400000
N = 100000
NPAD = 100352
NW = 32
E_PER_W = E // NW
BLK = 2000
NBLK = E_PER_W // BLK
VEC = 16
SUB = BLK // VEC
UNROLL = 5
NBUF = 4


def _sc_partial_sums(x, index):
  mesh = plsc.VectorSubcoreMesh(core_axis_name="c", subcore_axis_name="s")

  @functools.partial(
      pl.kernel,
      mesh=mesh,
      out_type=jax.ShapeDtypeStruct((NW, NPAD), jnp.float32),
      compiler_params=pltpu.CompilerParams(needs_layout_passes=False),
      scratch_types=[
          pltpu.VMEM((NPAD,), jnp.float32),
          [pltpu.VMEM((BLK,), jnp.float32)] * NBUF,
          [pltpu.VMEM((BLK,), jnp.int32)] * NBUF,
          [pltpu.SemaphoreType.DMA] * NBUF,
          [pltpu.SemaphoreType.DMA] * NBUF,
      ],
  )
  def k(x_hbm, idx_hbm, out_hbm, acc, xb, ib, sx, si):
    cid = lax.axis_index("c")
    sid = lax.axis_index("s")
    wid = sid * 2 + cid
    base = wid * E_PER_W

    zeros = jnp.zeros((VEC,), jnp.float32)

    @plsc.parallel_loop(0, NPAD // VEC, unroll=8)
    def _zero(i):
      acc[pl.ds(i * VEC, VEC)] = zeros

    def make_copies(blk, slot):
      off = base + blk * BLK
      cpx = pltpu.make_async_copy(
          x_hbm.at[pl.ds(off, BLK)], xb[slot], sx[slot])
      cpi = pltpu.make_async_copy(
          idx_hbm.at[pl.ds(off, BLK)], ib[slot], si[slot])
      return cpx, cpi

    def start(blk, slot):
      cpx, cpi = make_copies(blk, slot)
      cpx.start()
      cpi.start()

    def wait(blk, slot):
      cpx, cpi = make_copies(blk, slot)
      cpx.wait()
      cpi.wait()

    lanebase = jnp.arange(VEC, dtype=jnp.int32) * SUB

    def process(slot):
      @plsc.parallel_loop(0, SUB, unroll=UNROLL)
      def _scatter(i):
        pos = lanebase + i
        idxv = plsc.load_gather(ib[slot], [pos])
        xv = plsc.load_gather(xb[slot], [pos])
        plsc.addupdate_scatter(acc, [idxv], xv)

    for slot in range(NBUF):
      start(slot, slot)
    for blk in range(NBLK):
      slot = blk % NBUF
      wait(blk, slot)
      process(slot)
      if blk + NBUF < NBLK:
        start(blk + NBUF, slot)

    pltpu.sync_copy(acc, out_hbm.at[wid])

  return k(x, index)


def _tc_reduce(partials):
  def body(p_ref, o_ref):
    o_ref[...] = jnp.sum(p_ref[...], axis=0)

  return pl.pallas_call(
      body,
      out_shape=jax.ShapeDtypeStruct((NPAD,), jnp.float32),
  )(partials)


@jax.jit
def kernel(x, index):
  partials = _sc_partial_sums(x, index)
  out = _tc_reduce(partials)
  return out[:N]
